# Initial kernel scaffold; baseline (speedup 1.0000x reference)
import jax
import jax.numpy as jnp
from jax.experimental import pallas as pl


def kernel(feat_items, feat_users, edges_ub_src, edges_ub_dst, edges_bu_src, edges_bu_dst):
    raise NotImplementedError("write your pallas kernel here")


# speedup vs baseline: 3.9715x; 3.9715x over previous
import functools

import jax
import jax.numpy as jnp
from jax import lax
from jax.experimental import pallas as pl
from jax.experimental.pallas import tpu as pltpu
from jax.experimental.pallas import tpu_sc as plsc

N = 10000
E = 320000
D = 128
TEM = 0.7
DROP = 0.01

NS = 16
CH = 128
CPT = 160
EPT = CH * CPT
EPAD = EPT * NS
SENT = N
ACC_R = 10240
RPT = ACC_R // NS
ORPT = 624
OREM = N - ORPT * NS
DCAP = 4096
DPT = DCAP // NS
DCH = DPT // CH

_mesh = plsc.VectorSubcoreMesh(core_axis_name="c", subcore_axis_name="s")


@functools.partial(
    pl.kernel,
    out_type=[jax.ShapeDtypeStruct((N, D), jnp.float32)] * 4,
    mesh=_mesh,
    scratch_types=[
        pltpu.VMEM((CH,), jnp.int32),
        pltpu.VMEM((CH,), jnp.int32),
        pltpu.VMEM((CH, D), jnp.float32),
        pltpu.VMEM_SHARED((ACC_R, D), jnp.float32),
        pltpu.SemaphoreType.DMA,
    ],
)
def _sc_msgpass(feat_u, feat_i, s_ub, d_ub, s_bu, d_bu,
                ds_ub, dd_ub, ds_bu, dd_bu,
                hpos_ub, hpos_bu, dump_ub, dump_bu,
                sidx, didx, rows, acc, sem):
    c = lax.axis_index("c")
    s = lax.axis_index("s")

    def run(feat, src, dst, dsrc, ddst, hpos, dump):
        def zrow(i, _):
            for j in range(D // 16):
                rows[i, pl.ds(j * 16, 16)] = jnp.zeros((16,), jnp.float32)
            return 0
        lax.fori_loop(0, CH, zrow, 0)

        def zcp(k, _):
            pltpu.sync_copy(rows, acc.at[pl.ds(s * RPT + k * CH, CH)])
            return 0
        lax.fori_loop(0, RPT // CH, zcp, 0)
        plsc.subcore_barrier()

        base = s * EPT

        def body(i, _):
            off = base + i * CH
            pltpu.sync_copy(src.at[pl.ds(off, CH)], sidx)
            pltpu.sync_copy(dst.at[pl.ds(off, CH)], didx)
            pltpu.async_copy(feat.at[sidx], rows, sem).wait()
            pltpu.sync_copy(rows, acc.at[didx], add=True)
            return 0
        lax.fori_loop(0, CPT, body, 0)
        plsc.subcore_barrier()

        pltpu.sync_copy(acc.at[pl.ds(s * ORPT, ORPT)],
                        hpos.at[pl.ds(s * ORPT, ORPT)])

        @pl.when(s == NS - 1)
        def _():
            pltpu.sync_copy(acc.at[pl.ds(ORPT * NS, OREM)],
                            hpos.at[pl.ds(ORPT * NS, OREM)])
        plsc.subcore_barrier()

        dbase = s * DPT

        def dbody(i, _):
            off = dbase + i * CH
            pltpu.sync_copy(dsrc.at[pl.ds(off, CH)], sidx)
            pltpu.sync_copy(ddst.at[pl.ds(off, CH)], didx)
            pltpu.async_copy(feat.at[sidx], rows, sem).wait()
            pltpu.sync_copy(rows, acc.at[didx], add=True)
            return 0
        lax.fori_loop(0, DCH, dbody, 0)
        plsc.subcore_barrier()

        pltpu.sync_copy(acc.at[pl.ds(s * ORPT, ORPT)],
                        dump.at[pl.ds(s * ORPT, ORPT)])

        @pl.when(s == NS - 1)
        def _():
            pltpu.sync_copy(acc.at[pl.ds(ORPT * NS, OREM)],
                            dump.at[pl.ds(ORPT * NS, OREM)])

    @pl.when(c == 0)
    def _():
        run(feat_u, s_ub, d_ub, ds_ub, dd_ub, hpos_ub, dump_ub)

    @pl.when(c == 1)
    def _():
        run(feat_i, s_bu, d_bu, ds_bu, dd_bu, hpos_bu, dump_bu)


def _loss_body(aub, dub, abu, dbu, out):
    def one(a, dmp):
        b = 2.0 * a - dmp
        num = jnp.sum(a * b, axis=1)
        na = jnp.sqrt(jnp.sum(a * a, axis=1))
        nb = jnp.sqrt(jnp.sum(b * b, axis=1))
        cos = num / (jnp.maximum(na, 1e-8) * jnp.maximum(nb, 1e-8))
        return jnp.log(jnp.sum(jnp.exp(cos / TEM)))

    out[0, 0] = one(aub[...], dub[...]) + one(abu[...], dbu[...])


_tc_loss = pl.pallas_call(
    _loss_body,
    out_shape=jax.ShapeDtypeStruct((1, 1), jnp.float32),
    out_specs=pl.BlockSpec(memory_space=pltpu.SMEM),
)


def kernel(feat_items, feat_users, edges_ub_src, edges_ub_dst,
           edges_bu_src, edges_bu_dst):
    i32 = jnp.int32
    su = edges_ub_src.astype(i32)
    du = edges_ub_dst.astype(i32)
    sb = edges_bu_src.astype(i32)
    db = edges_bu_dst.astype(i32)

    pad = EPAD - E
    zpad = jnp.zeros((pad,), i32)
    spad = jnp.full((pad,), SENT, i32)
    su_p = jnp.concatenate([su, zpad])
    du_p = jnp.concatenate([du, spad])
    sb_p = jnp.concatenate([sb, zpad])
    db_p = jnp.concatenate([db, spad])

    drop_key = jax.random.key(42)
    k_ub, k_bu = jax.random.split(drop_key)
    keep_ub = jax.random.bernoulli(k_ub, p=1.0 - DROP, shape=(E,))
    keep_bu = jax.random.bernoulli(k_bu, p=1.0 - DROP, shape=(E,))
    ids_ub = jnp.where(~keep_ub, size=DCAP, fill_value=E)[0].astype(i32)
    ids_bu = jnp.where(~keep_bu, size=DCAP, fill_value=E)[0].astype(i32)
    ds_ub = jnp.take(su_p, ids_ub)
    dd_ub = jnp.take(du_p, ids_ub)
    ds_bu = jnp.take(sb_p, ids_bu)
    dd_bu = jnp.take(db_p, ids_bu)

    hpos_ub, hpos_bu, dump_ub, dump_bu = _sc_msgpass(
        feat_users, feat_items, su_p, du_p, sb_p, db_p,
        ds_ub, dd_ub, ds_bu, dd_bu)

    loss = _tc_loss(hpos_ub, dump_ub, hpos_bu, dump_bu)[0, 0]
    return hpos_ub, hpos_bu, loss


# speedup vs baseline: 5.2325x; 1.3175x over previous
import functools

import jax
import jax.numpy as jnp
from jax import lax
from jax.experimental import pallas as pl
from jax.experimental.pallas import tpu as pltpu
from jax.experimental.pallas import tpu_sc as plsc

N = 10000
E = 320000
D = 128
TEM = 0.7
DROP = 0.01

NS = 16
CH = 128
CPT = 160
EPT = CH * CPT
EPAD = EPT * NS
SENT = N
ACC_R = 10112
RPT = ACC_R // NS
IB = 32
NBLK = CPT // IB
NB = 2
ORPT = 624
OREM = N - ORPT * NS
DCAP = 4096
DPT = DCAP // NS
DCH = DPT // CH

_mesh = plsc.VectorSubcoreMesh(core_axis_name="c", subcore_axis_name="s")


@functools.partial(
    pl.kernel,
    out_type=[jax.ShapeDtypeStruct((N, D), jnp.float32)] * 4,
    mesh=_mesh,
    scratch_types=[
        pltpu.VMEM((IB, CH), jnp.int32),
        pltpu.VMEM((IB, CH), jnp.int32),
        pltpu.VMEM((CH,), jnp.int32),
        pltpu.VMEM((CH,), jnp.int32),
        [pltpu.VMEM((CH, D), jnp.float32) for _ in range(NB)],
        pltpu.VMEM_SHARED((ACC_R, D), jnp.float32),
        [pltpu.SemaphoreType.DMA for _ in range(NB)],
        [pltpu.SemaphoreType.DMA for _ in range(NB)],
    ],
)
def _sc_msgpass(feat_u, feat_i, s_ub, d_ub, s_bu, d_bu,
                ds_ub, dd_ub, ds_bu, dd_bu,
                hpos_ub, hpos_bu, dump_ub, dump_bu,
                sidx, didx, dsidx, ddidx, rows, acc, gsem, ssem):
    c = lax.axis_index("c")
    s = lax.axis_index("s")

    def run(feat, src, dst, dsrc, ddst, hpos, dump):
        def zrow(i, _):
            for j in range(D // 16):
                rows[0][i, pl.ds(j * 16, 16)] = jnp.zeros((16,), jnp.float32)
            return 0
        lax.fori_loop(0, CH, zrow, 0)

        def zcp(k, _):
            pltpu.sync_copy(rows[0], acc.at[pl.ds(s * RPT + k * CH, CH)])
            return 0
        lax.fori_loop(0, RPT // CH, zcp, 0)
        pltpu.sync_copy(rows[0].at[pl.ds(0, RPT - (RPT // CH) * CH)],
                        acc.at[pl.ds(s * RPT + (RPT // CH) * CH,
                                     RPT - (RPT // CH) * CH)])
        plsc.subcore_barrier()

        def gstart(i, b):
            pltpu.async_copy(feat.at[sidx.at[i]], rows[b], gsem[b])

        def sstart(i, b):
            pltpu.async_copy(rows[b], acc.at[didx.at[i]], ssem[b], add=True)

        def gwait(b):
            pltpu.make_async_copy(feat.at[sidx.at[0]], rows[b], gsem[b]).wait()

        def swait(b):
            pltpu.make_async_copy(rows[b], acc.at[didx.at[0]], ssem[b]).wait()

        def blk_body(blk, _):
            row0 = s * CPT + blk * IB
            pltpu.sync_copy(src.at[pl.ds(row0, IB)], sidx)
            pltpu.sync_copy(dst.at[pl.ds(row0, IB)], didx)
            gstart(0, 0)

            def body(k, _):
                for b in range(NB):
                    j = k * NB + b
                    nb = (b + 1) % NB

                    @pl.when(j >= NB - 1)
                    def _():
                        swait(nb)

                    @pl.when(j + 1 < IB)
                    def _():
                        gstart(j + 1, nb)

                    gwait(b)
                    sstart(j, b)
                return 0
            lax.fori_loop(0, IB // NB, body, 0)
            for b in ((IB - i) % NB for i in range(NB - 1, 0, -1)):
                swait(b)
            return 0
        lax.fori_loop(0, NBLK, blk_body, 0)
        plsc.subcore_barrier()

        pltpu.sync_copy(acc.at[pl.ds(s * ORPT, ORPT)],
                        hpos.at[pl.ds(s * ORPT, ORPT)])

        @pl.when(s == NS - 1)
        def _():
            pltpu.sync_copy(acc.at[pl.ds(ORPT * NS, OREM)],
                            hpos.at[pl.ds(ORPT * NS, OREM)])
        plsc.subcore_barrier()

        for ic in range(DCH):
            off = s * DPT + ic * CH
            pltpu.sync_copy(dsrc.at[pl.ds(off, CH)], dsidx)
            pltpu.sync_copy(ddst.at[pl.ds(off, CH)], ddidx)
            pltpu.async_copy(feat.at[dsidx], rows[0], gsem[0]).wait()
            pltpu.sync_copy(rows[0], acc.at[ddidx], add=True)
        plsc.subcore_barrier()

        pltpu.sync_copy(acc.at[pl.ds(s * ORPT, ORPT)],
                        dump.at[pl.ds(s * ORPT, ORPT)])

        @pl.when(s == NS - 1)
        def _():
            pltpu.sync_copy(acc.at[pl.ds(ORPT * NS, OREM)],
                            dump.at[pl.ds(ORPT * NS, OREM)])

    @pl.when(c == 0)
    def _():
        run(feat_u, s_ub, d_ub, ds_ub, dd_ub, hpos_ub, dump_ub)

    @pl.when(c == 1)
    def _():
        run(feat_i, s_bu, d_bu, ds_bu, dd_bu, hpos_bu, dump_bu)


def _loss_body(aub, dub, abu, dbu, out):
    def one(a, dmp):
        b = 2.0 * a - dmp
        num = jnp.sum(a * b, axis=1)
        na = jnp.sqrt(jnp.sum(a * a, axis=1))
        nb = jnp.sqrt(jnp.sum(b * b, axis=1))
        cos = num / (jnp.maximum(na, 1e-8) * jnp.maximum(nb, 1e-8))
        return jnp.log(jnp.sum(jnp.exp(cos / TEM)))

    out[0, 0] = one(aub[...], dub[...]) + one(abu[...], dbu[...])


_tc_loss = pl.pallas_call(
    _loss_body,
    out_shape=jax.ShapeDtypeStruct((1, 1), jnp.float32),
    out_specs=pl.BlockSpec(memory_space=pltpu.SMEM),
)


def kernel(feat_items, feat_users, edges_ub_src, edges_ub_dst,
           edges_bu_src, edges_bu_dst):
    i32 = jnp.int32
    su = edges_ub_src.astype(i32)
    du = edges_ub_dst.astype(i32)
    sb = edges_bu_src.astype(i32)
    db = edges_bu_dst.astype(i32)

    pad = EPAD - E
    zpad = jnp.zeros((pad,), i32)
    spad = jnp.full((pad,), SENT, i32)
    su_p = jnp.concatenate([su, zpad])
    du_p = jnp.concatenate([du, spad])
    sb_p = jnp.concatenate([sb, zpad])
    db_p = jnp.concatenate([db, spad])

    drop_key = jax.random.key(42)
    k_ub, k_bu = jax.random.split(drop_key)
    keep_ub = jax.random.bernoulli(k_ub, p=1.0 - DROP, shape=(E,))
    keep_bu = jax.random.bernoulli(k_bu, p=1.0 - DROP, shape=(E,))
    ids_ub = jnp.where(~keep_ub, size=DCAP, fill_value=E)[0].astype(i32)
    ids_bu = jnp.where(~keep_bu, size=DCAP, fill_value=E)[0].astype(i32)
    ds_ub = jnp.take(su_p, ids_ub)
    dd_ub = jnp.take(du_p, ids_ub)
    ds_bu = jnp.take(sb_p, ids_bu)
    dd_bu = jnp.take(db_p, ids_bu)

    sh2 = (NS * CPT, CH)
    hpos_ub, hpos_bu, dump_ub, dump_bu = _sc_msgpass(
        feat_users, feat_items,
        su_p.reshape(sh2), du_p.reshape(sh2),
        sb_p.reshape(sh2), db_p.reshape(sh2),
        ds_ub, dd_ub, ds_bu, dd_bu)

    loss = _tc_loss(hpos_ub, dump_ub, hpos_bu, dump_bu)[0, 0]
    return hpos_ub, hpos_bu, loss


# speedup vs baseline: 6.0053x; 1.1477x over previous
import functools

import jax
import jax.numpy as jnp
from jax import lax
from jax.experimental import pallas as pl
from jax.experimental.pallas import tpu as pltpu
from jax.experimental.pallas import tpu_sc as plsc

N = 10000
E = 320000
D = 128
TEM = 0.7
DROP = 0.01

NS = 16
CH = 128
CPT = 159
EPT = CH * CPT
EPAD = EPT * NS
SENT = N
ACC_R = 10016
RPT = 632
ORPT = 624
OREM = N - ORPT * NS
DCH = 2
DPT = DCH * CH
DCAP = DPT * NS
NSL = 3

_mesh = plsc.VectorSubcoreMesh(core_axis_name="c", subcore_axis_name="s")


@functools.partial(
    pl.kernel,
    out_type=[jax.ShapeDtypeStruct((N, D), jnp.float32)] * 4,
    mesh=_mesh,
    scratch_types=[
        [pltpu.VMEM((CH,), jnp.int32) for _ in range(NSL)],
        [pltpu.VMEM((CH,), jnp.int32) for _ in range(NSL)],
        [pltpu.VMEM((CH, D), jnp.float32) for _ in range(NSL)],
        pltpu.VMEM_SHARED((ACC_R, D), jnp.float32),
        [pltpu.SemaphoreType.DMA for _ in range(NSL)],
        [pltpu.SemaphoreType.DMA for _ in range(NSL)],
        [pltpu.SemaphoreType.DMA for _ in range(NSL)],
    ],
)
def _sc_msgpass(feat_u, feat_i, s_ub, d_ub, s_bu, d_bu,
                ds_ub, dd_ub, ds_bu, dd_bu,
                hpos_ub, hpos_bu, dump_ub, dump_bu,
                sidx, didx, stage, acc, ssem, dsem, gsem):
    c = lax.axis_index("c")
    s = lax.axis_index("s")

    def run(feat, src, dst, dsrc, ddst, hpos, dump):
        base = s * EPT

        def istart(j, b):
            pltpu.async_copy(src.at[pl.ds(base + j * CH, CH)], sidx[b],
                             ssem[b])
            pltpu.async_copy(dst.at[pl.ds(base + j * CH, CH)], didx[b],
                             dsem[b])

        def iwait(b):
            pltpu.make_async_copy(src.at[pl.ds(0, CH)], sidx[b],
                                  ssem[b]).wait()
            pltpu.make_async_copy(dst.at[pl.ds(0, CH)], didx[b],
                                  dsem[b]).wait()

        def gstart(b):
            pltpu.async_copy(feat.at[sidx[b]], stage[b], gsem[b])

        def gwait(b):
            pltpu.make_async_copy(feat.at[sidx[b]], stage[b], gsem[b]).wait()

        def zrow(i, _):
            for g in range(D // 16):
                stage[0][i, pl.ds(g * 16, 16)] = jnp.zeros((16,), jnp.float32)
            return 0
        lax.fori_loop(0, CH, zrow, 0)
        for t in range(4):
            pltpu.sync_copy(stage[0], acc.at[pl.ds(s * RPT + t * CH, CH)])

        @pl.when(s < NS - 1)
        def _():
            pltpu.sync_copy(stage[0].at[pl.ds(0, 120)],
                            acc.at[pl.ds(s * RPT + 512, 120)])

        @pl.when(s == NS - 1)
        def _():
            pltpu.sync_copy(stage[0].at[pl.ds(0, 24)],
                            acc.at[pl.ds(s * RPT + 512, 24)])
        plsc.subcore_barrier()

        for k in range(NSL):
            istart(k, k)
        iwait(0)
        gstart(0)
        iwait(1)
        gstart(1)

        def body(k, _):
            for b in range(NSL):
                j = k * NSL + b
                nb = (b + 2) % NSL

                @pl.when(j + 2 < CPT)
                def _():
                    iwait(nb)
                    gstart(nb)

                gwait(b)
                pltpu.sync_copy(stage[b], acc.at[didx[b]], add=True)

                @pl.when(j + NSL < CPT)
                def _():
                    istart(j + NSL, b)
            return 0
        lax.fori_loop(0, CPT // NSL, body, 0)
        plsc.subcore_barrier()

        pltpu.sync_copy(acc.at[pl.ds(s * ORPT, ORPT)],
                        hpos.at[pl.ds(s * ORPT, ORPT)])

        @pl.when(s == NS - 1)
        def _():
            pltpu.sync_copy(acc.at[pl.ds(ORPT * NS, OREM)],
                            hpos.at[pl.ds(ORPT * NS, OREM)])
        plsc.subcore_barrier()

        for ic in range(DCH):
            off = s * DPT + ic * CH
            pltpu.sync_copy(dsrc.at[pl.ds(off, CH)], sidx[0])
            pltpu.sync_copy(ddst.at[pl.ds(off, CH)], didx[0])
            pltpu.async_copy(feat.at[sidx[0]], stage[0], gsem[0]).wait()
            pltpu.sync_copy(stage[0], acc.at[didx[0]], add=True)
        plsc.subcore_barrier()

        pltpu.sync_copy(acc.at[pl.ds(s * ORPT, ORPT)],
                        dump.at[pl.ds(s * ORPT, ORPT)])

        @pl.when(s == NS - 1)
        def _():
            pltpu.sync_copy(acc.at[pl.ds(ORPT * NS, OREM)],
                            dump.at[pl.ds(ORPT * NS, OREM)])

    @pl.when(c == 0)
    def _():
        run(feat_u, s_ub, d_ub, ds_ub, dd_ub, hpos_ub, dump_ub)

    @pl.when(c == 1)
    def _():
        run(feat_i, s_bu, d_bu, ds_bu, dd_bu, hpos_bu, dump_bu)


def _loss_body(aub, dub, abu, dbu, out):
    def one(a, dmp):
        b = 2.0 * a - dmp
        num = jnp.sum(a * b, axis=1)
        na = jnp.sqrt(jnp.sum(a * a, axis=1))
        nb = jnp.sqrt(jnp.sum(b * b, axis=1))
        cos = num / (jnp.maximum(na, 1e-8) * jnp.maximum(nb, 1e-8))
        return jnp.log(jnp.sum(jnp.exp(cos / TEM)))

    out[0, 0] = one(aub[...], dub[...]) + one(abu[...], dbu[...])


_tc_loss = pl.pallas_call(
    _loss_body,
    out_shape=jax.ShapeDtypeStruct((1, 1), jnp.float32),
    out_specs=pl.BlockSpec(memory_space=pltpu.SMEM),
)


def kernel(feat_items, feat_users, edges_ub_src, edges_ub_dst,
           edges_bu_src, edges_bu_dst):
    i32 = jnp.int32
    su = edges_ub_src.astype(i32)
    du = edges_ub_dst.astype(i32)
    sb = edges_bu_src.astype(i32)
    db = edges_bu_dst.astype(i32)

    pad = EPAD - E
    zpad = jnp.zeros((pad,), i32)
    spad = jnp.full((pad,), SENT, i32)
    su_p = jnp.concatenate([su, zpad])
    du_p = jnp.concatenate([du, spad])
    sb_p = jnp.concatenate([sb, zpad])
    db_p = jnp.concatenate([db, spad])

    drop_key = jax.random.key(42)
    k_ub, k_bu = jax.random.split(drop_key)
    keep_ub = jax.random.bernoulli(k_ub, p=1.0 - DROP, shape=(E,))
    keep_bu = jax.random.bernoulli(k_bu, p=1.0 - DROP, shape=(E,))
    ids_ub = jnp.where(~keep_ub, size=DCAP, fill_value=E)[0].astype(i32)
    ids_bu = jnp.where(~keep_bu, size=DCAP, fill_value=E)[0].astype(i32)
    ds_ub = jnp.take(su_p, ids_ub)
    dd_ub = jnp.take(du_p, ids_ub)
    ds_bu = jnp.take(sb_p, ids_bu)
    dd_bu = jnp.take(db_p, ids_bu)

    hpos_ub, hpos_bu, dump_ub, dump_bu = _sc_msgpass(
        feat_users, feat_items, su_p, du_p, sb_p, db_p,
        ds_ub, dd_ub, ds_bu, dd_bu)

    loss = _tc_loss(hpos_ub, dump_ub, hpos_bu, dump_bu)[0, 0]
    return hpos_ub, hpos_bu, loss
